# baseline (speedup 1.0000x reference)
import functools

import jax
import jax.numpy as jnp
from jax.experimental import pallas as pl
from jax.experimental.pallas import tpu as pltpu

_E = 8
_K = 2
_BS = 2048


def _default_dot(x, w):
    return jnp.dot(x.astype(jnp.bfloat16), w.astype(jnp.bfloat16),
                   preferred_element_type=jnp.float32)


def _router_body(x_ref, w_ref, b_ref, gates_ref, idx_ref, tw_ref, aux_ref,
                 imp_ref, load_ref):
    i = pl.program_id(0)

    @pl.when(i == 0)
    def _init():
        imp_ref[...] = jnp.zeros_like(imp_ref)
        load_ref[...] = jnp.zeros_like(load_ref)

    x = x_ref[...]
    logits = _default_dot(x, w_ref[...])
    logits = logits + b_ref[...]
    lt = logits.T
    m = jnp.max(lt, axis=0, keepdims=True)
    ex = jnp.exp(lt - m)
    s = jnp.sum(ex, axis=0, keepdims=True)
    p = ex / s
    gates_ref[...] = p

    imp_ref[...] += jnp.sum(p, axis=1, keepdims=True)
    load_ref[...] += jnp.sum((p > 0).astype(jnp.float32), axis=1,
                             keepdims=True)

    iota = jax.lax.broadcasted_iota(jnp.int32, p.shape, 0)
    m1 = jnp.max(p, axis=0, keepdims=True)
    i1 = jnp.min(jnp.where(p == m1, iota, _E), axis=0, keepdims=True)
    pm = jnp.where(iota == i1, -jnp.inf, p)
    m2 = jnp.max(pm, axis=0, keepdims=True)
    i2 = jnp.min(jnp.where(pm == m2, iota, _E), axis=0, keepdims=True)
    idx_ref[...] = jnp.concatenate([i1, i2], axis=0)
    denom = m1 + m2 + 1e-9
    tw_ref[...] = jnp.concatenate([m1 / denom, m2 / denom], axis=0)

    @pl.when(i == pl.num_programs(0) - 1)
    def _finish():
        x8 = imp_ref[...] * load_ref[...]
        mean = jnp.sum(x8, keepdims=True) * (1.0 / _E)
        var = jnp.sum((x8 - mean) ** 2, keepdims=True) * (1.0 / (_E - 1))
        aux_ref[...] = var * 0.01


@functools.partial(jax.jit, static_argnames=())
def _router(flat, W, b2):
    n, h = flat.shape
    grid = (n // _BS,)
    gates_t, idx_t, tw_t, aux = pl.pallas_call(
        _router_body,
        grid=grid,
        in_specs=[
            pl.BlockSpec((_BS, h), lambda i: (i, 0)),
            pl.BlockSpec((h, _E), lambda i: (0, 0)),
            pl.BlockSpec((1, _E), lambda i: (0, 0)),
        ],
        out_specs=[
            pl.BlockSpec((_E, _BS), lambda i: (0, i)),
            pl.BlockSpec((_K, _BS), lambda i: (0, i)),
            pl.BlockSpec((_K, _BS), lambda i: (0, i)),
            pl.BlockSpec((1, 1), lambda i: (0, 0)),
        ],
        out_shape=[
            jax.ShapeDtypeStruct((_E, n), jnp.float32),
            jax.ShapeDtypeStruct((_K, n), jnp.int32),
            jax.ShapeDtypeStruct((_K, n), jnp.float32),
            jax.ShapeDtypeStruct((1, 1), jnp.float32),
        ],
        scratch_shapes=[
            pltpu.VMEM((_E, 1), jnp.float32),
            pltpu.VMEM((_E, 1), jnp.float32),
        ],
    )(flat, W, b2)
    return gates_t, idx_t, tw_t, aux


def kernel(tokens, W, b):
    g, s, h = tokens.shape
    e = W.shape[1]
    flat = tokens.reshape(g * s, h)
    gates_t, idx_t, tw_t, aux = _router(flat, W, b.reshape(1, e))
    return (idx_t.T.reshape(g, s, _K), tw_t.T.reshape(g, s, _K), aux[0, 0],
            gates_t.T.reshape(g, s, e))


# speedup vs baseline: 1.0593x; 1.0593x over previous
import functools

import jax
import jax.numpy as jnp
from jax.experimental import pallas as pl
from jax.experimental.pallas import tpu as pltpu

_E = 8
_K = 2
_BS = 4096


def _default_dot(x, w):
    return jnp.dot(x.astype(jnp.bfloat16), w.astype(jnp.bfloat16),
                   preferred_element_type=jnp.float32)


def _router_body(x_ref, w_ref, b_ref, gates_ref, idx_ref, tw_ref, aux_ref,
                 imp_ref, load_ref):
    i = pl.program_id(0)

    @pl.when(i == 0)
    def _init():
        imp_ref[...] = jnp.zeros_like(imp_ref)
        load_ref[...] = jnp.zeros_like(load_ref)

    x = x_ref[...]
    logits = _default_dot(x, w_ref[...])
    logits = logits + b_ref[...]
    lt = logits.T
    m = jnp.max(lt, axis=0, keepdims=True)
    ex = jnp.exp(lt - m)
    s = jnp.sum(ex, axis=0, keepdims=True)
    p = ex / s
    gates_ref[...] = p

    imp_ref[...] += jnp.sum(p, axis=1, keepdims=True)
    load_ref[...] += jnp.sum((p > 0).astype(jnp.float32), axis=1,
                             keepdims=True)

    iota = jax.lax.broadcasted_iota(jnp.int32, p.shape, 0)
    m1 = jnp.max(p, axis=0, keepdims=True)
    i1 = jnp.min(jnp.where(p == m1, iota, _E), axis=0, keepdims=True)
    pm = jnp.where(iota == i1, -jnp.inf, p)
    m2 = jnp.max(pm, axis=0, keepdims=True)
    i2 = jnp.min(jnp.where(pm == m2, iota, _E), axis=0, keepdims=True)
    idx_ref[...] = jnp.concatenate([i1, i2], axis=0)
    denom = m1 + m2 + 1e-9
    tw_ref[...] = jnp.concatenate([m1 / denom, m2 / denom], axis=0)

    @pl.when(i == pl.num_programs(0) - 1)
    def _finish():
        x8 = imp_ref[...] * load_ref[...]
        mean = jnp.sum(x8, keepdims=True) * (1.0 / _E)
        var = jnp.sum((x8 - mean) ** 2, keepdims=True) * (1.0 / (_E - 1))
        aux_ref[...] = var * 0.01


@functools.partial(jax.jit, static_argnames=())
def _router(flat, W, b2):
    n, h = flat.shape
    grid = (n // _BS,)
    gates_t, idx_t, tw_t, aux = pl.pallas_call(
        _router_body,
        grid=grid,
        in_specs=[
            pl.BlockSpec((_BS, h), lambda i: (i, 0)),
            pl.BlockSpec((h, _E), lambda i: (0, 0)),
            pl.BlockSpec((1, _E), lambda i: (0, 0)),
        ],
        out_specs=[
            pl.BlockSpec((_E, _BS), lambda i: (0, i)),
            pl.BlockSpec((_K, _BS), lambda i: (0, i)),
            pl.BlockSpec((_K, _BS), lambda i: (0, i)),
            pl.BlockSpec((1, 1), lambda i: (0, 0)),
        ],
        out_shape=[
            jax.ShapeDtypeStruct((_E, n), jnp.float32),
            jax.ShapeDtypeStruct((_K, n), jnp.int32),
            jax.ShapeDtypeStruct((_K, n), jnp.float32),
            jax.ShapeDtypeStruct((1, 1), jnp.float32),
        ],
        scratch_shapes=[
            pltpu.VMEM((_E, 1), jnp.float32),
            pltpu.VMEM((_E, 1), jnp.float32),
        ],
    )(flat, W, b2)
    return gates_t, idx_t, tw_t, aux


def kernel(tokens, W, b):
    g, s, h = tokens.shape
    e = W.shape[1]
    flat = tokens.reshape(g * s, h)
    gates_t, idx_t, tw_t, aux = _router(flat, W, b.reshape(1, e))
    return (idx_t.T.reshape(g, s, _K), tw_t.T.reshape(g, s, _K), aux[0, 0],
            gates_t.T.reshape(g, s, e))
